# baseline (speedup 1.0000x reference)
import jax
import jax.numpy as jnp
from jax.experimental import pallas as pl

_NUM_CATEGORIES = 256
_BATCH_BLOCK = 16
_PERIOD = 3200
_TPERIOD = 16


def _one_hot_block(idx_ref, out_ref):
    v = idx_ref[...]
    s = jax.lax.broadcasted_iota(jnp.int32, (1, 1, _PERIOD), 2)
    p = s // 200
    t = jax.lax.broadcasted_iota(jnp.int32, (1, _TPERIOD, 1), 1)
    target = p + t * _TPERIOD
    out_ref[...] = (v[:, None, :] == target).astype(jnp.float32)


def kernel(indices):
    batch, seq = indices.shape
    bblk = _BATCH_BLOCK
    idx_tiled = jnp.tile(indices, (1, _PERIOD // seq))
    out_flat = pl.pallas_call(
        _one_hot_block,
        grid=(batch // bblk,),
        in_specs=[pl.BlockSpec((bblk, _PERIOD), lambda i: (i, 0))],
        out_specs=pl.BlockSpec((bblk, _TPERIOD, _PERIOD), lambda i: (i, 0, 0)),
        out_shape=jax.ShapeDtypeStruct((batch, _TPERIOD, _PERIOD), jnp.float32),
    )(idx_tiled)
    return out_flat.reshape(batch, _NUM_CATEGORIES, seq)


# speedup vs baseline: 1.5332x; 1.5332x over previous
import jax
import jax.numpy as jnp
from jax.experimental import pallas as pl

_NUM_CATEGORIES = 256
_BATCH_BLOCK = 64


def _one_hot_block(idx_ref, out_ref):
    idx = idx_ref[...]
    cat = jax.lax.broadcasted_iota(
        jnp.int32, (idx.shape[0], _NUM_CATEGORIES, idx.shape[1]), 1)
    out_ref[...] = (idx[:, None, :] == cat).astype(jnp.float32)


def kernel(indices):
    batch, seq = indices.shape
    bblk = _BATCH_BLOCK
    grid = (batch // bblk,)
    return pl.pallas_call(
        _one_hot_block,
        grid=grid,
        in_specs=[pl.BlockSpec((bblk, seq), lambda i: (i, 0))],
        out_specs=pl.BlockSpec((bblk, _NUM_CATEGORIES, seq), lambda i: (i, 0, 0)),
        out_shape=jax.ShapeDtypeStruct((batch, _NUM_CATEGORIES, seq), jnp.float32),
    )(indices)
